# baseline (speedup 1.0000x reference)
import jax
import jax.numpy as jnp
from jax import lax
from jax.experimental import pallas as pl
from jax.experimental.pallas import tpu as pltpu

FINAL_ITER = 200000
THRESHOLD = 0.6

_ROWS = 128
_COLS = 100000
_BLOCK_ROWS = 8
_NBLK = _ROWS // _BLOCK_ROWS

_KS0 = 0
_KS1 = 12345
_KS2 = _KS0 ^ _KS1 ^ 0x1BD11BDA
_ROT0 = (13, 15, 26, 6)
_ROT1 = (17, 29, 16, 24)
_KS = (_KS0, _KS1, _KS2)


def _threefry_bits(n):
    x0 = jnp.uint32(_KS[0])
    x1 = n + jnp.uint32(_KS[1])
    rotations = (_ROT0, _ROT1)
    for i_round in range(5):
        for d in rotations[i_round % 2]:
            x0 = x0 + x1
            x1 = (x1 << jnp.uint32(d)) | (x1 >> jnp.uint32(32 - d))
            x1 = x0 ^ x1
        x0 = x0 + jnp.uint32(_KS[(i_round + 1) % 3])
        x1 = x1 + jnp.uint32((_KS[(i_round + 2) % 3] + i_round + 1) & 0xFFFFFFFF)
    return x0 ^ x1


def _body(t_hbm, y_hbm, thr_ref, o_hbm,
          t0, t1, y0, y1, o0, o1, st0, st1, sy0, sy1, so0, so1):
    thr = thr_ref[0]
    nbase = (
        lax.broadcasted_iota(jnp.uint32, (_BLOCK_ROWS, _COLS), 0)
        * jnp.uint32(_COLS)
        + lax.broadcasted_iota(jnp.uint32, (_BLOCK_ROWS, _COLS), 1)
    )

    def in_t(b, buf, sem):
        return pltpu.make_async_copy(
            t_hbm.at[pl.ds(b * _BLOCK_ROWS, _BLOCK_ROWS), :], buf, sem)

    def in_y(b, buf, sem):
        return pltpu.make_async_copy(
            y_hbm.at[pl.ds(b * _BLOCK_ROWS, _BLOCK_ROWS), :], buf, sem)

    def out_o(b, buf, sem):
        return pltpu.make_async_copy(
            buf, o_hbm.at[pl.ds(b * _BLOCK_ROWS, _BLOCK_ROWS), :], sem)

    def compute(b, tbuf, ybuf, obuf):
        n = nbase + (b * (_BLOCK_ROWS * _COLS)).astype(jnp.uint32)
        mask = _threefry_bits(n) < thr
        obuf[...] = jnp.where(mask, ybuf[...], tbuf[...])

    in_t(0, t0, st0).start()
    in_y(0, y0, sy0).start()
    in_t(0, t0, st0).wait()
    in_y(0, y0, sy0).wait()

    def step(j, carry):
        compute(j, t0, y0, o0)
        return carry

    lax.fori_loop(0, _NBLK, step, 0)
    out_o(0, o0, so0).start()
    out_o(0, o0, so0).wait()
    out_o(1, o1, so1).start()
    out_o(1, o1, so1).wait()


def kernel(target, y, now_iter):
    k = 1.0
    c = (k - THRESHOLD) / FINAL_ITER
    sampling_prob = jnp.maximum(THRESHOLD, k - c * now_iter)
    p = 1.0 - sampling_prob
    thr = (jnp.ceil(p * 8388608.0).astype(jnp.uint32) << 9).reshape(1)

    buf = pltpu.VMEM((_BLOCK_ROWS, _COLS), jnp.float32)
    return pl.pallas_call(
        _body,
        in_specs=[
            pl.BlockSpec(memory_space=pl.ANY),
            pl.BlockSpec(memory_space=pl.ANY),
            pl.BlockSpec(memory_space=pltpu.SMEM),
        ],
        out_specs=pl.BlockSpec(memory_space=pl.ANY),
        out_shape=jax.ShapeDtypeStruct((_ROWS, _COLS), jnp.float32),
        scratch_shapes=[buf, buf, buf, buf, buf, buf]
        + [pltpu.SemaphoreType.DMA] * 6,
    )(target, y, thr)


# speedup vs baseline: 1.0018x; 1.0018x over previous
import jax
import jax.numpy as jnp
from jax import lax
from jax.experimental import pallas as pl
from jax.experimental.pallas import tpu as pltpu

FINAL_ITER = 200000
THRESHOLD = 0.6

_ROWS = 128
_COLS = 100000
_BLOCK_ROWS = 8
_NBLK = _ROWS // _BLOCK_ROWS

_KS0 = 0
_KS1 = 12345
_KS2 = _KS0 ^ _KS1 ^ 0x1BD11BDA
_ROT0 = (13, 15, 26, 6)
_ROT1 = (17, 29, 16, 24)
_KS = (_KS0, _KS1, _KS2)


def _threefry_bits(n):
    x0 = jnp.uint32(_KS[0])
    x1 = n + jnp.uint32(_KS[1])
    rotations = (_ROT0, _ROT1)
    for i_round in range(5):
        for d in rotations[i_round % 2]:
            x0 = x0 + x1
            x1 = (x1 << jnp.uint32(d)) | (x1 >> jnp.uint32(32 - d))
            x1 = x0 ^ x1
        x0 = x0 + jnp.uint32(_KS[(i_round + 1) % 3])
        x1 = x1 + jnp.uint32((_KS[(i_round + 2) % 3] + i_round + 1) & 0xFFFFFFFF)
    return x0 ^ x1


def _body(t_hbm, y_hbm, thr_ref, o_hbm,
          t0, t1, y0, y1, o0, o1, st0, st1, sy0, sy1, so0, so1):
    thr = thr_ref[0]
    nbase = (
        lax.broadcasted_iota(jnp.uint32, (_BLOCK_ROWS, _COLS), 0)
        * jnp.uint32(_COLS)
        + lax.broadcasted_iota(jnp.uint32, (_BLOCK_ROWS, _COLS), 1)
    )

    def in_t(b, buf, sem):
        return pltpu.make_async_copy(
            t_hbm.at[pl.ds(b * _BLOCK_ROWS, _BLOCK_ROWS), :], buf, sem)

    def in_y(b, buf, sem):
        return pltpu.make_async_copy(
            y_hbm.at[pl.ds(b * _BLOCK_ROWS, _BLOCK_ROWS), :], buf, sem)

    def out_o(b, buf, sem):
        return pltpu.make_async_copy(
            buf, o_hbm.at[pl.ds(b * _BLOCK_ROWS, _BLOCK_ROWS), :], sem)

    def compute(b, tbuf, ybuf, obuf):
        n = nbase + (b * (_BLOCK_ROWS * _COLS)).astype(jnp.uint32)
        mask = _threefry_bits(n) < thr
        obuf[...] = jnp.where(mask, ybuf[...], tbuf[...])

    in_t(0, t0, st0).start()
    in_y(0, y0, sy0).start()

    def step(j, carry):
        b0 = 2 * j
        b1 = 2 * j + 1

        in_t(b1, t1, st1).start()
        in_y(b1, y1, sy1).start()
        in_t(b0, t0, st0).wait()
        in_y(b0, y0, sy0).wait()

        @pl.when(j >= 1)
        def _():
            out_o(b0 - 2, o0, so0).wait()

        compute(b0, t0, y0, o0)
        out_o(b0, o0, so0).start()

        @pl.when(j < _NBLK // 2 - 1)
        def _():
            in_t(b1 + 1, t0, st0).start()
            in_y(b1 + 1, y0, sy0).start()

        in_t(b1, t1, st1).wait()
        in_y(b1, y1, sy1).wait()

        @pl.when(j >= 1)
        def _():
            out_o(b1 - 2, o1, so1).wait()

        compute(b1, t1, y1, o1)
        out_o(b1, o1, so1).start()
        return carry

    lax.fori_loop(0, _NBLK // 2, step, 0)
    out_o(_NBLK - 2, o0, so0).wait()
    out_o(_NBLK - 1, o1, so1).wait()


def kernel(target, y, now_iter):
    k = 1.0
    c = (k - THRESHOLD) / FINAL_ITER
    sampling_prob = jnp.maximum(THRESHOLD, k - c * now_iter)
    p = 1.0 - sampling_prob
    thr = (jnp.ceil(p * 8388608.0).astype(jnp.uint32) << 9).reshape(1)

    buf = pltpu.VMEM((_BLOCK_ROWS, _COLS), jnp.float32)
    return pl.pallas_call(
        _body,
        in_specs=[
            pl.BlockSpec(memory_space=pl.ANY),
            pl.BlockSpec(memory_space=pl.ANY),
            pl.BlockSpec(memory_space=pltpu.SMEM),
        ],
        out_specs=pl.BlockSpec(memory_space=pl.ANY),
        out_shape=jax.ShapeDtypeStruct((_ROWS, _COLS), jnp.float32),
        scratch_shapes=[buf, buf, buf, buf, buf, buf]
        + [pltpu.SemaphoreType.DMA] * 6,
    )(target, y, thr)
